# baseline (speedup 1.0000x reference)
import jax
import jax.numpy as jnp
from jax.experimental import pallas as pl
from jax.experimental.pallas import tpu as pltpu

_ALPHA = 1e-05
_NF = 128


def _sampler_body(t0_ref, t1_ref, ts_ref, w_ref, u_ref, tr_ref, out_ref):
    blk, nc = ts_ref.shape
    ts = ts_ref[...]
    w = w_ref[...] + _ALPHA
    u = u_ref[...]
    tr = tr_ref[...]
    t0 = t0_ref[...]
    t1 = t1_ref[...]

    row = jax.lax.broadcasted_iota(jnp.int32, (nc, nc), 0)
    col = jax.lax.broadcasted_iota(jnp.int32, (nc, nc), 1)
    tri = (row <= col).astype(jnp.float32)
    cdf = jax.lax.dot(w, tri, preferred_element_type=jnp.float32)
    uq = u * cdf[:, nc - 1:nc]

    mids = (ts[:, 1:] + ts[:, :-1]) * 0.5
    lo = jnp.concatenate([t0, mids], axis=1)
    up = jnp.concatenate([mids, t1], axis=1)
    wid = up - lo
    zcol = jnp.zeros_like(t0)
    dlo = jnp.concatenate([lo[:, 1:] - lo[:, :-1], zcol], axis=1)
    dwid = jnp.concatenate([wid[:, 1:] - wid[:, :-1], zcol], axis=1)

    acc_lo = jnp.broadcast_to(lo[:, 0:1], (blk, _NF))
    acc_wid = jnp.broadcast_to(wid[:, 0:1], (blk, _NF))
    for j in range(nc):
        m = uq > cdf[:, j:j + 1]
        acc_lo = acc_lo + jnp.where(m, dlo[:, j:j + 1], 0.0)
        acc_wid = acc_wid + jnp.where(m, dwid[:, j:j + 1], 0.0)

    v = acc_lo + acc_wid * tr

    lane = jax.lax.broadcasted_iota(jnp.int32, (blk, _NF), 1)
    k = 2
    while k <= _NF:
        d = k // 2
        while d >= 1:
            partner = jnp.where((lane & d) == 0,
                                pltpu.roll(v, _NF - d, 1),
                                pltpu.roll(v, d, 1))
            take_min = ((lane & k) == 0) == ((lane & d) == 0)
            v = jnp.where(take_min,
                          jnp.minimum(v, partner),
                          jnp.maximum(v, partner))
            d //= 2
        k *= 2

    out_ref[...] = v


def kernel(t0, t1, ts, weights, n_samples):
    del n_samples
    b, r, nc, _ = ts.shape
    n = b * r
    blk = min(64, n)

    t0r = t0.reshape(n, 1)
    t1r = t1.reshape(n, 1)
    tsr = ts.reshape(n, nc)
    wr = weights.reshape(n, nc)
    u = jax.random.uniform(jax.random.key(1), (n, _NF), dtype=weights.dtype)
    tr = jax.random.uniform(jax.random.key(2), (b, r, _NF, 1),
                            dtype=ts.dtype).reshape(n, _NF)

    out = pl.pallas_call(
        _sampler_body,
        grid=(n // blk,),
        in_specs=[
            pl.BlockSpec((blk, 1), lambda i: (i, 0)),
            pl.BlockSpec((blk, 1), lambda i: (i, 0)),
            pl.BlockSpec((blk, nc), lambda i: (i, 0)),
            pl.BlockSpec((blk, nc), lambda i: (i, 0)),
            pl.BlockSpec((blk, _NF), lambda i: (i, 0)),
            pl.BlockSpec((blk, _NF), lambda i: (i, 0)),
        ],
        out_specs=pl.BlockSpec((blk, _NF), lambda i: (i, 0)),
        out_shape=jax.ShapeDtypeStruct((n, _NF), jnp.float32),
        compiler_params=pltpu.CompilerParams(
            dimension_semantics=("arbitrary",),
        ),
    )(t0r, t1r, tsr, wr, u, tr)
    return out.reshape(b, r, _NF, 1)


# speedup vs baseline: 3.7677x; 3.7677x over previous
import jax
import jax.numpy as jnp
from jax.experimental import pallas as pl
from jax.experimental.pallas import tpu as pltpu

_ALPHA = 1e-05
_NF = 128


def _sampler_body(t0_ref, t1_ref, ts_ref, w_ref, u_ref, tr_ref, out_ref):
    nc, rblk = ts_ref.shape
    ts = ts_ref[...]
    w = w_ref[...] + _ALPHA
    u = u_ref[...]
    tr = tr_ref[...]
    t0 = t0_ref[...]
    t1 = t1_ref[...]

    row = jax.lax.broadcasted_iota(jnp.int32, (nc, nc), 0)
    col = jax.lax.broadcasted_iota(jnp.int32, (nc, nc), 1)
    tri = (col <= row).astype(jnp.float32)
    cdf = jax.lax.dot(tri, w, preferred_element_type=jnp.float32)
    uq = u * cdf[nc - 1:nc, :]

    mids = (ts[1:, :] + ts[:-1, :]) * 0.5
    lo = jnp.concatenate([t0, mids], axis=0)
    up = jnp.concatenate([mids, t1], axis=0)
    wid = up - lo
    zrow = jnp.zeros_like(t0)
    dlo = jnp.concatenate([lo[1:, :] - lo[:-1, :], zrow], axis=0)
    dwid = jnp.concatenate([wid[1:, :] - wid[:-1, :], zrow], axis=0)

    acc_lo = jnp.broadcast_to(lo[0:1, :], (_NF, rblk))
    acc_wid = jnp.broadcast_to(wid[0:1, :], (_NF, rblk))
    for j in range(nc):
        m = uq > cdf[j:j + 1, :]
        acc_lo = acc_lo + jnp.where(m, dlo[j:j + 1, :], 0.0)
        acc_wid = acc_wid + jnp.where(m, dwid[j:j + 1, :], 0.0)

    vt = acc_lo + acc_wid * tr

    eye = (jax.lax.broadcasted_iota(jnp.int32, (_NF, _NF), 0) ==
           jax.lax.broadcasted_iota(jnp.int32, (_NF, _NF), 1)).astype(jnp.float32)
    v = jax.lax.dot_general(vt, eye, (((0,), (0,)), ((), ())),
                            preferred_element_type=jnp.float32)

    lane = jax.lax.broadcasted_iota(jnp.int32, (rblk, _NF), 1)
    k = 2
    while k <= _NF:
        d = k // 2
        while d >= 1:
            partner = jnp.where((lane & d) == 0,
                                pltpu.roll(v, _NF - d, 1),
                                pltpu.roll(v, d, 1))
            take_min = ((lane & k) == 0) == ((lane & d) == 0)
            v = jnp.where(take_min,
                          jnp.minimum(v, partner),
                          jnp.maximum(v, partner))
            d //= 2
        k *= 2

    out_ref[...] = v


def kernel(t0, t1, ts, weights, n_samples):
    del n_samples
    b, r, nc, _ = ts.shape
    n = b * r
    rblk = min(256, n)

    t0r = t0.reshape(1, n)
    t1r = t1.reshape(1, n)
    tsr = ts.reshape(n, nc).T
    wr = weights.reshape(n, nc).T
    u = jax.random.uniform(jax.random.key(1), (n, _NF), dtype=weights.dtype).T
    tr = jax.random.uniform(jax.random.key(2), (b, r, _NF, 1),
                            dtype=ts.dtype).reshape(n, _NF).T

    out = pl.pallas_call(
        _sampler_body,
        grid=(n // rblk,),
        in_specs=[
            pl.BlockSpec((1, rblk), lambda i: (0, i)),
            pl.BlockSpec((1, rblk), lambda i: (0, i)),
            pl.BlockSpec((nc, rblk), lambda i: (0, i)),
            pl.BlockSpec((nc, rblk), lambda i: (0, i)),
            pl.BlockSpec((_NF, rblk), lambda i: (0, i)),
            pl.BlockSpec((_NF, rblk), lambda i: (0, i)),
        ],
        out_specs=pl.BlockSpec((rblk, _NF), lambda i: (i, 0)),
        out_shape=jax.ShapeDtypeStruct((n, _NF), jnp.float32),
        compiler_params=pltpu.CompilerParams(
            dimension_semantics=("arbitrary",),
        ),
    )(t0r, t1r, tsr, wr, u, tr)
    return out.reshape(b, r, _NF, 1)
